# baseline (speedup 1.0000x reference)
import functools

import jax
import jax.numpy as jnp
from jax.experimental import pallas as pl
from jax.experimental.pallas import tpu as pltpu


def _pick_block(n, candidates):
    for c in candidates:
        if n % c == 0:
            return c
    return n


def _ln(t, g, b):
    m = jnp.mean(t, axis=-1, keepdims=True)
    v = jnp.mean((t - m) * (t - m), axis=-1, keepdims=True)
    return (t - m) * jax.lax.rsqrt(v + 1e-5) * g + b


def _split16(t):
    hi = t.astype(jnp.bfloat16)
    lo = (t - hi.astype(jnp.float32)).astype(jnp.bfloat16)
    return hi, lo


def _p0_body(x_ref, win_ref, p0_ref):
    p0_ref[...] = jnp.dot(x_ref[...], win_ref[...],
                          preferred_element_type=jnp.float32)


def _p1_body(a_ref, p0_ref, wh_ref, g0_ref, b0_ref,
             h0_ref, p1h_ref, p1l_ref, b16_ref, cs_ref, *, c):
    a = a_ref[...]
    b16_ref[...] = (a - c).astype(jnp.bfloat16)
    t = jnp.dot(a, p0_ref[...], preferred_element_type=jnp.float32)
    h = _ln(jnp.maximum(t, 0.0), g0_ref[...], b0_ref[...])
    h0_ref[...] = h
    p1 = jnp.dot(h, wh_ref[...], preferred_element_type=jnp.float32)
    p1h_ref[...], p1l_ref[...] = _split16(p1)

    @pl.when(pl.program_id(0) == 0)
    def _():
        cs_ref[...] = jnp.zeros_like(cs_ref)

    cs_ref[...] += jnp.sum(p1, axis=0, keepdims=True)


def _corr_dot(b16, ph_ref, pl_ref, cs_ref, c):
    t = jnp.dot(b16, ph_ref[...], preferred_element_type=jnp.float32)
    t += jnp.dot(b16, pl_ref[...], preferred_element_type=jnp.float32)
    return t + c * cs_ref[...]


def _p2_body(b_ref, p1h_ref, p1l_ref, cs1_ref, h0_ref, wml_ref, g1_ref,
             b1_ref, p2h_ref, p2l_ref, cs2_ref, *, c):
    t = _corr_dot(b_ref[...], p1h_ref, p1l_ref, cs1_ref, c)
    h1 = _ln(jnp.maximum(t, 0.0), g1_ref[...], b1_ref[...]) + h0_ref[...]
    p2 = jnp.dot(h1, wml_ref[...], preferred_element_type=jnp.float32)
    p2h_ref[...], p2l_ref[...] = _split16(p2)

    @pl.when(pl.program_id(0) == 0)
    def _():
        cs2_ref[...] = jnp.zeros_like(cs2_ref)

    cs2_ref[...] += jnp.sum(p2, axis=0, keepdims=True)


def _p3_body(b_ref, p2h_ref, p2l_ref, cs2_ref, eps_ref,
             gm_ref, bm_ref, gl_ref, bl_ref, z_ref, *, c, h2):
    t = _corr_dot(b_ref[...], p2h_ref, p2l_ref, cs2_ref, c)
    mean = _ln(t[:, :h2], gm_ref[...], bm_ref[...])
    logstd = _ln(t[:, h2:], gl_ref[...], bl_ref[...])
    z_ref[...] = eps_ref[...] * jnp.exp(logstd) + mean


def _p4_body(zr_ref, zc_ref, out_ref):
    s = jax.lax.dot_general(
        zr_ref[...], zc_ref[...],
        dimension_numbers=(((1,), (1,)), ((), ())),
        preferred_element_type=jnp.float32,
    )
    out_ref[...] = jax.nn.sigmoid(s)


def kernel(x, adj_norm, W_in, W_h, W_mean, W_logstd,
           g0, b0, g1, b1, gm, bm, gl, bl, eps):
    n, d = x.shape
    h1d = W_in.shape[1]
    h2 = W_mean.shape[1]
    c = 0.5 / n
    br = _pick_block(n, (400, 200, 100, 16))
    nb = n // br

    g0r, b0r = g0.reshape(1, -1), b0.reshape(1, -1)
    g1r, b1r = g1.reshape(1, -1), b1.reshape(1, -1)
    gmr, bmr = gm.reshape(1, -1), bm.reshape(1, -1)
    glr, blr = gl.reshape(1, -1), bl.reshape(1, -1)
    wml = jnp.concatenate([W_mean, W_logstd], axis=1)

    row_spec = pl.BlockSpec((br, n), lambda i: (i, 0))
    full = lambda shape: pl.BlockSpec(shape, lambda i: tuple(0 for _ in shape))
    out_row = lambda w: pl.BlockSpec((br, w), lambda i: (i, 0))

    p0 = pl.pallas_call(
        _p0_body,
        in_specs=[pl.BlockSpec((n, d), lambda: (0, 0)),
                  pl.BlockSpec((d, h1d), lambda: (0, 0))],
        out_specs=pl.BlockSpec((n, h1d), lambda: (0, 0)),
        out_shape=jax.ShapeDtypeStruct((n, h1d), jnp.float32),
    )(x, W_in)

    h0, p1h, p1l, b16, cs1 = pl.pallas_call(
        functools.partial(_p1_body, c=c),
        grid=(nb,),
        in_specs=[row_spec, full((n, h1d)), full((h1d, h1d)),
                  full((1, h1d)), full((1, h1d))],
        out_specs=[out_row(h1d), out_row(h1d), out_row(h1d), row_spec,
                   full((1, h1d))],
        out_shape=[jax.ShapeDtypeStruct((n, h1d), jnp.float32),
                   jax.ShapeDtypeStruct((n, h1d), jnp.bfloat16),
                   jax.ShapeDtypeStruct((n, h1d), jnp.bfloat16),
                   jax.ShapeDtypeStruct((n, n), jnp.bfloat16),
                   jax.ShapeDtypeStruct((1, h1d), jnp.float32)],
    )(adj_norm, p0, W_h, g0r, b0r)

    p2h, p2l, cs2 = pl.pallas_call(
        functools.partial(_p2_body, c=c),
        grid=(nb,),
        in_specs=[row_spec, full((n, h1d)), full((n, h1d)), full((1, h1d)),
                  out_row(h1d), full((h1d, 2 * h2)),
                  full((1, h1d)), full((1, h1d))],
        out_specs=[out_row(2 * h2), out_row(2 * h2), full((1, 2 * h2))],
        out_shape=[jax.ShapeDtypeStruct((n, 2 * h2), jnp.bfloat16),
                   jax.ShapeDtypeStruct((n, 2 * h2), jnp.bfloat16),
                   jax.ShapeDtypeStruct((1, 2 * h2), jnp.float32)],
    )(b16, p1h, p1l, cs1, h0, wml, g1r, b1r)

    z = pl.pallas_call(
        functools.partial(_p3_body, c=c, h2=h2),
        grid=(nb,),
        in_specs=[row_spec, full((n, 2 * h2)), full((n, 2 * h2)),
                  full((1, 2 * h2)), out_row(h2),
                  full((1, h2)), full((1, h2)), full((1, h2)), full((1, h2))],
        out_specs=out_row(h2),
        out_shape=jax.ShapeDtypeStruct((n, h2), jnp.float32),
    )(b16, p2h, p2l, cs2, eps, gmr, bmr, glr, blr)

    out = pl.pallas_call(
        _p4_body,
        grid=(nb,),
        in_specs=[out_row(h2), pl.BlockSpec((n, h2), lambda i: (0, 0))],
        out_specs=row_spec,
        out_shape=jax.ShapeDtypeStruct((n, n), jnp.float32),
    )(z, z)
    return out


# speedup vs baseline: 1.2287x; 1.2287x over previous
import functools

import jax
import jax.numpy as jnp
from jax.experimental import pallas as pl
from jax.experimental.pallas import tpu as pltpu


def _pick_block(n, candidates):
    for c in candidates:
        if n % c == 0:
            return c
    return n


def _ln(t, g, b):
    m = jnp.mean(t, axis=-1, keepdims=True)
    v = jnp.mean((t - m) * (t - m), axis=-1, keepdims=True)
    return (t - m) * jax.lax.rsqrt(v + 1e-5) * g + b


def _p0_body(x_ref, win_ref, p0_ref):
    p0_ref[...] = jnp.dot(x_ref[...], win_ref[...],
                          preferred_element_type=jnp.float32)


def _p1_body(a_ref, p0_ref, wh_ref, g0_ref, b0_ref,
             h0_ref, p1_ref, b16_ref, cs_ref, *, c):
    a = a_ref[...]
    b16_ref[...] = (a - c).astype(jnp.bfloat16)
    t = jnp.dot(a, p0_ref[...], preferred_element_type=jnp.float32)
    h = _ln(jnp.maximum(t, 0.0), g0_ref[...], b0_ref[...])
    h0_ref[...] = h
    p1 = jnp.dot(h, wh_ref[...], preferred_element_type=jnp.float32)
    p1_ref[...] = p1.astype(jnp.bfloat16)

    @pl.when(pl.program_id(0) == 0)
    def _():
        cs_ref[...] = jnp.zeros_like(cs_ref)

    cs_ref[...] += jnp.sum(p1, axis=0, keepdims=True)


def _p23_body(b_ref, p1_ref, cs1_ref, h0_ref, wml_ref, g1_ref, b1_ref,
              eps_ref, gm_ref, bm_ref, gl_ref, bl_ref,
              z_ref, p2_scr, cs2_scr, *, c, h2, br):
    ph = pl.program_id(0)
    i = pl.program_id(1)
    rhs = jnp.where(ph == 0, p1_ref[...], p2_scr[...])
    cs = jnp.where(ph == 0, cs1_ref[...], cs2_scr[...])
    t = jnp.dot(b_ref[...], rhs, preferred_element_type=jnp.float32) + c * cs

    @pl.when(ph == 0)
    def _():
        h1 = _ln(jnp.maximum(t, 0.0), g1_ref[...], b1_ref[...]) + h0_ref[...]
        p2 = jnp.dot(h1, wml_ref[...], preferred_element_type=jnp.float32)
        p2_scr[pl.ds(i * br, br), :] = p2.astype(jnp.bfloat16)
        cs2_scr[...] = jnp.where(
            i == 0, jnp.zeros_like(cs2_scr), cs2_scr[...]
        ) + jnp.sum(p2, axis=0, keepdims=True)
        z_ref[...] = t[:, :h2]

    @pl.when(ph == 1)
    def _():
        mean = _ln(t[:, :h2], gm_ref[...], bm_ref[...])
        logstd = _ln(t[:, h2:], gl_ref[...], bl_ref[...])
        z_ref[...] = eps_ref[...] * jnp.exp(logstd) + mean


def _p4_body(zr_ref, zc_ref, out_ref):
    s = jax.lax.dot_general(
        zr_ref[...], zc_ref[...],
        dimension_numbers=(((1,), (1,)), ((), ())),
        preferred_element_type=jnp.float32,
    )
    out_ref[...] = jax.nn.sigmoid(s)


def kernel(x, adj_norm, W_in, W_h, W_mean, W_logstd,
           g0, b0, g1, b1, gm, bm, gl, bl, eps):
    n, d = x.shape
    h1d = W_in.shape[1]
    h2 = W_mean.shape[1]
    c = 0.5 / n
    br = _pick_block(n, (400, 200, 100, 16))
    nb = n // br

    g0r, b0r = g0.reshape(1, -1), b0.reshape(1, -1)
    g1r, b1r = g1.reshape(1, -1), b1.reshape(1, -1)
    gmr, bmr = gm.reshape(1, -1), bm.reshape(1, -1)
    glr, blr = gl.reshape(1, -1), bl.reshape(1, -1)
    wml = jnp.concatenate([W_mean, W_logstd], axis=1)

    row_spec = pl.BlockSpec((br, n), lambda i: (i, 0))
    full = lambda shape: pl.BlockSpec(shape, lambda i: tuple(0 for _ in shape))
    out_row = lambda w: pl.BlockSpec((br, w), lambda i: (i, 0))
    row_spec2 = pl.BlockSpec((br, n), lambda p, i: (i, 0))
    full2 = lambda shape: pl.BlockSpec(shape, lambda p, i: tuple(0 for _ in shape))
    row2 = lambda w: pl.BlockSpec((br, w), lambda p, i: (i, 0))

    p0 = pl.pallas_call(
        _p0_body,
        in_specs=[pl.BlockSpec((n, d), lambda: (0, 0)),
                  pl.BlockSpec((d, h1d), lambda: (0, 0))],
        out_specs=pl.BlockSpec((n, h1d), lambda: (0, 0)),
        out_shape=jax.ShapeDtypeStruct((n, h1d), jnp.float32),
    )(x, W_in)

    h0, p1, b16, cs1 = pl.pallas_call(
        functools.partial(_p1_body, c=c),
        grid=(nb,),
        in_specs=[row_spec, full((n, h1d)), full((h1d, h1d)),
                  full((1, h1d)), full((1, h1d))],
        out_specs=[out_row(h1d), out_row(h1d), row_spec, full((1, h1d))],
        out_shape=[jax.ShapeDtypeStruct((n, h1d), jnp.float32),
                   jax.ShapeDtypeStruct((n, h1d), jnp.bfloat16),
                   jax.ShapeDtypeStruct((n, n), jnp.bfloat16),
                   jax.ShapeDtypeStruct((1, h1d), jnp.float32)],
    )(adj_norm, p0, W_h, g0r, b0r)

    z = pl.pallas_call(
        functools.partial(_p23_body, c=c, h2=h2, br=br),
        grid=(2, nb),
        in_specs=[row_spec2, full2((n, h1d)), full2((1, h1d)), row2(h1d),
                  full2((h1d, 2 * h2)), full2((1, h1d)), full2((1, h1d)),
                  row2(h2), full2((1, h2)), full2((1, h2)),
                  full2((1, h2)), full2((1, h2))],
        out_specs=row2(h2),
        out_shape=jax.ShapeDtypeStruct((n, h2), jnp.float32),
        scratch_shapes=[pltpu.VMEM((n, 2 * h2), jnp.bfloat16),
                        pltpu.VMEM((1, 2 * h2), jnp.float32)],
    )(b16, p1, cs1, h0, wml, g1r, b1r, eps, gmr, bmr, glr, blr)

    out = pl.pallas_call(
        _p4_body,
        grid=(nb,),
        in_specs=[out_row(h2), pl.BlockSpec((n, h2), lambda i: (0, 0))],
        out_specs=row_spec,
        out_shape=jax.ShapeDtypeStruct((n, n), jnp.float32),
        compiler_params=pltpu.CompilerParams(
            dimension_semantics=("parallel",)),
    )(z, z)
    return out
